# baseline (speedup 1.0000x reference)
import functools

import jax
import jax.numpy as jnp
from jax import lax
from jax.experimental import pallas as pl
from jax.experimental.pallas import tpu as pltpu
from jax.experimental.pallas import tpu_sc as plsc

_VOCAB = 1000000
_DIM = 64
_B = 4096
_L = 200

_NC = 2
_NS = 16
_NW = _NC * _NS

_T1_ROWS = _VOCAB // 2
_KW = 256
_NBLK = _VOCAB // _KW
_TAIL = _VOCAB - _NBLK * _KW
_BCHUNK = _B // _NW

_PARAMS = pltpu.CompilerParams(
    use_tc_tiling_on_sc=True, needs_layout_passes=False
)


def _wid():
    return lax.axis_index("s") * _NC + lax.axis_index("c")


def _iota16():
    return lax.iota(jnp.int32, 16)


def _build_format():
    mesh = plsc.VectorSubcoreMesh(core_axis_name="c", subcore_axis_name="s")

    @functools.partial(
        pl.kernel,
        mesh=mesh,
        out_type=jax.ShapeDtypeStruct((_T1_ROWS, 128), jnp.float32),
        scratch_types=[
            pltpu.VMEM((3, 64, _KW), jnp.float32),
            pltpu.VMEM((3, 128, 128), jnp.float32),
        ]
        + [pltpu.SemaphoreType.DMA] * 7,
        compiler_params=_PARAMS,
    )
    def fmt(tT, tail, T1, A, Bv, ra0, ra1, ra2, wb0, wb1, wb2, ts):
        rsem = (ra0, ra1, ra2)
        wsem = (wb0, wb1, wb2)
        w = _wid()

        @pl.when(w == 0)
        def _():
            pltpu.async_copy(tail, T1.at[pl.ds(_NBLK * 128, 32)], ts).wait()

        cnt = (_NBLK - w + _NW - 1) // _NW

        def read_start(i, buf):
            blk = w + i * _NW
            pltpu.async_copy(tT.at[:, pl.ds(blk * _KW, _KW)], A.at[buf], rsem[buf])

        def read_wait(buf):
            pltpu.make_async_copy(
                tT.at[:, pl.ds(0, _KW)], A.at[buf], rsem[buf]
            ).wait()

        def write_start(i, buf):
            blk = w + i * _NW
            pltpu.async_copy(Bv.at[buf], T1.at[pl.ds(blk * 128, 128)], wsem[buf])

        def write_wait(buf):
            pltpu.make_async_copy(
                Bv.at[buf], T1.at[pl.ds(0, 128)], wsem[buf]
            ).wait()

        read_start(0, 0)

        @pl.when(cnt > 1)
        def _():
            read_start(1, 1)

        def body3(i3, carry):
            for p in (0, 1, 2):
                i = i3 * 3 + p

                @pl.when(i < cnt)
                def _(i=i, p=p):
                    read_wait(p)

                    @pl.when(i + 2 < cnt)
                    def _():
                        read_start(i + 2, (p + 2) % 3)

                    @pl.when(i >= 3)
                    def _():
                        write_wait(p)

                    src = A.at[p]
                    dst = Bv.at[p]

                    @plsc.parallel_loop(0, 16, unroll=4)
                    def ds_loop(s, src=src, dst=dst):
                        sv = jnp.broadcast_to(s, (16,)).astype(jnp.int32)
                        rotv = jnp.bitwise_and(_iota16() + sv, 15)
                        for c0 in range(0, 128, 16):
                            rv = rotv + (c0 % 64)
                            cdv = rotv + c0
                            hh = c0 // 64
                            for u0 in range(0, 128, 16):
                                cv = _iota16() * 2 + (2 * u0 + hh)
                                uv = _iota16() + u0
                                v = plsc.load_gather(src, [rv, cv])
                                plsc.store_scatter(dst, [uv, cdv], v)

                    write_start(i, p)

            return carry

        lax.fori_loop(0, (cnt + 2) // 3, body3, 0)

        for s in (0, 1, 2):
            for back in (1, 2, 3):
                @pl.when((cnt >= back) & (lax.rem(cnt - back + 3, 3) == s))
                def _(s=s):
                    write_wait(s)

    return fmt


def _build_gather():
    mesh = plsc.VectorSubcoreMesh(core_axis_name="c", subcore_axis_name="s")

    @functools.partial(
        pl.kernel,
        mesh=mesh,
        out_type=jax.ShapeDtypeStruct((_L, _DIM, _B), jnp.float32),
        scratch_types=[
            pltpu.VMEM((_L, 128), jnp.int32),
            pltpu.VMEM((4, 128), jnp.int32),
            pltpu.VMEM((4, 128), jnp.int32),
            pltpu.VMEM((4, 128, 128), jnp.float32),
            pltpu.VMEM((4, 64, 128), jnp.float32),
        ]
        + [pltpu.SemaphoreType.DMA] * 9,
        compiler_params=_PARAMS,
    )
    def gat(xT, T1, out3, xt, jv, cbv, G, Bv,
            xs, gs0, gs1, gs2, gs3, os0, os1, os2, os3):
        gsem = (gs0, gs1, gs2, gs3)
        osem = (os0, os1, os2, os3)
        w = _wid()
        b0 = w * _BCHUNK

        def prep(l, q):
            for g in range(8):
                r = xt[l, pl.ds(16 * g, 16)]
                jv[q, pl.ds(16 * g, 16)] = jnp.right_shift(r, 1)
                cbv[q, pl.ds(16 * g, 16)] = jnp.left_shift(
                    jnp.bitwise_and(r, 1), 6
                )

        def gather_start(q):
            pltpu.async_copy(T1.at[jv.at[q]], G.at[q], gsem[q])

        def gather_wait(q):
            pltpu.make_async_copy(T1.at[jv.at[q]], G.at[q], gsem[q]).wait()

        def out_start(l, buf):
            pltpu.async_copy(
                Bv.at[buf], out3.at[l, :, pl.ds(b0, 128)], osem[buf]
            )

        def out_wait(buf):
            pltpu.make_async_copy(
                Bv.at[buf], out3.at[0, :, pl.ds(0, 128)], osem[buf]
            ).wait()

        pltpu.async_copy(xT.at[:, pl.ds(b0, 128)], xt, xs).wait()
        for q in range(3):
            prep(q, q)
            gather_start(q)

        def body4(i4, carry):
            for q in (0, 1, 2, 3):
                l = i4 * 4 + q
                p = q & 1

                @pl.when(l + 3 < _L)
                def _(l=l, q=q):
                    prep(l + 3, (q + 3) % 4)
                    gather_start((q + 3) % 4)

                gather_wait(q)

                @pl.when(l >= 4)
                def _(q=q):
                    out_wait(q)

                src = G.at[q]
                dst = Bv.at[q]
                cbls = [cbv[q, pl.ds(c0, 16)] for c0 in range(0, 128, 16)]

                @plsc.parallel_loop(0, 16, unroll=4)
                def ds_loop(s, src=src, dst=dst, cbls=cbls):
                    sv = jnp.broadcast_to(s, (16,)).astype(jnp.int32)
                    rotv = jnp.bitwise_and(_iota16() + sv, 15)
                    for d0 in range(0, 64, 16):
                        rdv = rotv + d0
                        for ci, c0 in enumerate(range(0, 128, 16)):
                            rv = _iota16() + c0
                            cv = cbls[ci] + rdv
                            v = plsc.load_gather(src, [rv, cv])
                            plsc.store_scatter(dst, [rdv, rv], v)

                out_start(l, q)

            return carry

        lax.fori_loop(0, _L // 4, body4, 0)

        for q in (0, 1, 2, 3):
            out_wait(q)

    return gat


_FMT = _build_format()
_GAT = _build_gather()


def kernel(x, table):
    tT = table.T
    tail = table[_NBLK * _KW:].reshape(_TAIL // 2, 128)
    t1 = _FMT(tT, tail)
    xT = x.T
    out3 = _GAT(xT, t1)
    return jnp.transpose(out3, (2, 0, 1))


# speedup vs baseline: 1.0842x; 1.0842x over previous
import functools

import jax
import jax.numpy as jnp
from jax import lax
from jax.experimental import pallas as pl
from jax.experimental.pallas import tpu as pltpu
from jax.experimental.pallas import tpu_sc as plsc

_VOCAB = 1000000
_DIM = 64
_B = 4096
_L = 200

_NC = 2
_NS = 16
_NW = _NC * _NS

_T1_ROWS = _VOCAB // 2
_KW = 256
_NBLK = _VOCAB // _KW
_TAIL = _VOCAB - _NBLK * _KW
_BCHUNK = _B // _NW

_PARAMS = pltpu.CompilerParams(
    use_tc_tiling_on_sc=True, needs_layout_passes=False
)


def _wid():
    return lax.axis_index("s") * _NC + lax.axis_index("c")


def _iota16():
    return lax.iota(jnp.int32, 16)


def _build_format():
    mesh = plsc.VectorSubcoreMesh(core_axis_name="c", subcore_axis_name="s")

    @functools.partial(
        pl.kernel,
        mesh=mesh,
        out_type=jax.ShapeDtypeStruct((_T1_ROWS, 128), jnp.float32),
        scratch_types=[
            pltpu.VMEM((3, 64, _KW), jnp.float32),
            pltpu.VMEM((3, 128, 128), jnp.float32),
        ]
        + [pltpu.SemaphoreType.DMA] * 7,
        compiler_params=_PARAMS,
    )
    def fmt(tT, tail, T1, A, Bv, ra0, ra1, ra2, wb0, wb1, wb2, ts):
        rsem = (ra0, ra1, ra2)
        wsem = (wb0, wb1, wb2)
        w = _wid()

        @pl.when(w == 0)
        def _():
            pltpu.async_copy(tail, T1.at[pl.ds(_NBLK * 128, 32)], ts).wait()

        cnt = (_NBLK - w + _NW - 1) // _NW

        def read_start(i, buf):
            blk = w + i * _NW
            pltpu.async_copy(tT.at[:, pl.ds(blk * _KW, _KW)], A.at[buf], rsem[buf])

        def read_wait(buf):
            pltpu.make_async_copy(
                tT.at[:, pl.ds(0, _KW)], A.at[buf], rsem[buf]
            ).wait()

        def write_start(i, buf):
            blk = w + i * _NW
            pltpu.async_copy(Bv.at[buf], T1.at[pl.ds(blk * 128, 128)], wsem[buf])

        def write_wait(buf):
            pltpu.make_async_copy(
                Bv.at[buf], T1.at[pl.ds(0, 128)], wsem[buf]
            ).wait()

        read_start(0, 0)

        @pl.when(cnt > 1)
        def _():
            read_start(1, 1)

        def body3(i3, carry):
            for p in (0, 1, 2):
                i = i3 * 3 + p

                @pl.when(i < cnt)
                def _(i=i, p=p):
                    read_wait(p)

                    @pl.when(i + 2 < cnt)
                    def _():
                        read_start(i + 2, (p + 2) % 3)

                    @pl.when(i >= 3)
                    def _():
                        write_wait(p)

                    src = A.at[p]
                    dst = Bv.at[p]

                    @plsc.parallel_loop(0, 16, unroll=4)
                    def ds_loop(s, src=src, dst=dst):
                        sv = jnp.broadcast_to(s, (16,)).astype(jnp.int32)
                        rotv = jnp.bitwise_and(_iota16() + sv, 15)
                        for c0 in range(0, 128, 16):
                            rv = rotv + (c0 % 64)
                            cdv = rotv + c0
                            hh = c0 // 64
                            for u0 in range(0, 128, 16):
                                cv = _iota16() * 2 + (2 * u0 + hh)
                                uv = _iota16() + u0
                                v = plsc.load_gather(src, [rv, cv])
                                plsc.store_scatter(dst, [uv, cdv], v)

                plsc.subcore_barrier()

                @pl.when(i < cnt)
                def _(i=i, p=p):
                    write_start(i, p)

            return carry

        lax.fori_loop(0, (cnt + 2) // 3, body3, 0)

        for s in (0, 1, 2):
            for back in (1, 2, 3):
                @pl.when((cnt >= back) & (lax.rem(cnt - back + 3, 3) == s))
                def _(s=s):
                    write_wait(s)

    return fmt


def _build_gather():
    mesh = plsc.VectorSubcoreMesh(core_axis_name="c", subcore_axis_name="s")

    @functools.partial(
        pl.kernel,
        mesh=mesh,
        out_type=jax.ShapeDtypeStruct((_L, _DIM, _B), jnp.float32),
        scratch_types=[
            pltpu.VMEM((_L, 128), jnp.int32),
            pltpu.VMEM((4, 128), jnp.int32),
            pltpu.VMEM((4, 128), jnp.int32),
            pltpu.VMEM((4, 128, 128), jnp.float32),
            pltpu.VMEM((4, 64, 128), jnp.float32),
        ]
        + [pltpu.SemaphoreType.DMA] * 9,
        compiler_params=_PARAMS,
    )
    def gat(xT, T1, out3, xt, jv, cbv, G, Bv,
            xs, gs0, gs1, gs2, gs3, os0, os1, os2, os3):
        gsem = (gs0, gs1, gs2, gs3)
        osem = (os0, os1, os2, os3)
        w = _wid()
        b0 = w * _BCHUNK

        def prep(l, q):
            for g in range(8):
                r = xt[l, pl.ds(16 * g, 16)]
                jv[q, pl.ds(16 * g, 16)] = jnp.right_shift(r, 1)
                cbv[q, pl.ds(16 * g, 16)] = jnp.left_shift(
                    jnp.bitwise_and(r, 1), 6
                )

        def gather_start(q):
            pltpu.async_copy(T1.at[jv.at[q]], G.at[q], gsem[q])

        def gather_wait(q):
            pltpu.make_async_copy(T1.at[jv.at[q]], G.at[q], gsem[q]).wait()

        def out_start(l, buf):
            pltpu.async_copy(
                Bv.at[buf], out3.at[l, :, pl.ds(b0, 128)], osem[buf]
            )

        def out_wait(buf):
            pltpu.make_async_copy(
                Bv.at[buf], out3.at[0, :, pl.ds(0, 128)], osem[buf]
            ).wait()

        pltpu.async_copy(xT.at[:, pl.ds(b0, 128)], xt, xs).wait()
        for q in range(3):
            prep(q, q)
            plsc.subcore_barrier()
            gather_start(q)

        def body4(i4, carry):
            for q in (0, 1, 2, 3):
                l = i4 * 4 + q
                p = q & 1

                @pl.when(l + 3 < _L)
                def _(l=l, q=q):
                    prep(l + 3, (q + 3) % 4)
                    plsc.subcore_barrier()
                    gather_start((q + 3) % 4)

                gather_wait(q)

                @pl.when(l >= 4)
                def _(q=q):
                    out_wait(q)

                src = G.at[q]
                dst = Bv.at[q]
                cbls = [cbv[q, pl.ds(c0, 16)] for c0 in range(0, 128, 16)]

                @plsc.parallel_loop(0, 16, unroll=4)
                def ds_loop(s, src=src, dst=dst, cbls=cbls):
                    sv = jnp.broadcast_to(s, (16,)).astype(jnp.int32)
                    rotv = jnp.bitwise_and(_iota16() + sv, 15)
                    for d0 in range(0, 64, 16):
                        rdv = rotv + d0
                        for ci, c0 in enumerate(range(0, 128, 16)):
                            rv = _iota16() + c0
                            cv = cbls[ci] + rdv
                            v = plsc.load_gather(src, [rv, cv])
                            plsc.store_scatter(dst, [rdv, rv], v)

                plsc.subcore_barrier()
                out_start(l, q)

            return carry

        lax.fori_loop(0, _L // 4, body4, 0)

        for q in (0, 1, 2, 3):
            out_wait(q)

    return gat


_FMT = _build_format()
_GAT = _build_gather()


def kernel(x, table):
    tT = table.T
    tail = table[_NBLK * _KW:].reshape(_TAIL // 2, 128)
    t1 = _FMT(tT, tail)
    xT = x.T
    out3 = _GAT(xT, t1)
    return jnp.transpose(out3, (2, 0, 1))
